# baseline (speedup 1.0000x reference)
import jax
import jax.numpy as jnp
from jax.experimental import pallas as pl
from jax.experimental.pallas import tpu as pltpu

NUM_HEADS = 16
NUM_KV_HEADS = 4
HEAD_DIM = 128
SCALE = 0.08838834764831845
BQ = 256
BK = 256


def _flash_body(q_ref, k_ref, v_ref, o_ref):
    qi = pl.program_id(2)
    q = q_ref[0, 0] * SCALE
    rows = jax.lax.broadcasted_iota(jnp.int32, (BQ, BK), 0) + qi * BQ

    def step(ki, carry):
        m, l, acc = carry
        kk = k_ref[0, 0, pl.ds(ki * BK, BK), :]
        s = jax.lax.dot_general(q, kk, (((1,), (1,)), ((), ())),
                                preferred_element_type=jnp.float32)
        cols = jax.lax.broadcasted_iota(jnp.int32, (BQ, BK), 1) + ki * BK
        s = jnp.where(rows >= cols, s, -1e30)
        m_new = jnp.maximum(m, s.max(axis=1, keepdims=True))
        p = jnp.exp(s - m_new)
        alpha = jnp.exp(m - m_new)
        vv = v_ref[0, 0, pl.ds(ki * BK, BK), :]
        acc = acc * alpha + jax.lax.dot_general(p, vv, (((1,), (0,)), ((), ())),
                                                preferred_element_type=jnp.float32)
        l = l * alpha + p.sum(axis=1, keepdims=True)
        return m_new, l, acc

    m0 = jnp.full((BQ, 1), -jnp.inf, jnp.float32)
    l0 = jnp.zeros((BQ, 1), jnp.float32)
    acc0 = jnp.zeros((BQ, HEAD_DIM), jnp.float32)
    m, l, acc = jax.lax.fori_loop(0, qi + 1, step, (m0, l0, acc0))
    o_ref[0, 0] = acc / l


def kernel(q, k, v, cu_seqlens_q):
    B = int(cu_seqlens_q.shape[0]) - 1
    T = q.shape[0]
    S = T // B
    rep = NUM_HEADS // NUM_KV_HEADS
    nq = S // BQ

    qb = q.reshape(B, S, NUM_HEADS, HEAD_DIM).transpose(0, 2, 1, 3)
    kb = k.reshape(B, S, NUM_KV_HEADS, HEAD_DIM).transpose(0, 2, 1, 3)
    vb = v.reshape(B, S, NUM_KV_HEADS, HEAD_DIM).transpose(0, 2, 1, 3)

    ob = pl.pallas_call(
        _flash_body,
        grid=(B, NUM_HEADS, nq),
        in_specs=[
            pl.BlockSpec((1, 1, BQ, HEAD_DIM), lambda b, h, i: (b, h, i, 0)),
            pl.BlockSpec((1, 1, S, HEAD_DIM), lambda b, h, i: (b, h // rep, 0, 0)),
            pl.BlockSpec((1, 1, S, HEAD_DIM), lambda b, h, i: (b, h // rep, 0, 0)),
        ],
        out_specs=pl.BlockSpec((1, 1, BQ, HEAD_DIM), lambda b, h, i: (b, h, i, 0)),
        out_shape=jax.ShapeDtypeStruct((B, NUM_HEADS, S, HEAD_DIM), jnp.float32),
        compiler_params=pltpu.CompilerParams(
            dimension_semantics=("parallel", "parallel", "arbitrary")),
    )(qb, kb, vb)

    return ob.transpose(0, 2, 1, 3).reshape(T, NUM_HEADS, HEAD_DIM)


# speedup vs baseline: 1.0720x; 1.0720x over previous
import jax
import jax.numpy as jnp
from jax.experimental import pallas as pl
from jax.experimental.pallas import tpu as pltpu

NUM_HEADS = 16
NUM_KV_HEADS = 4
HEAD_DIM = 128
SCALE = 0.08838834764831845
BQ = 256
NEG = -1e30


def _flash_body(q_ref, k_ref, v_ref, o_ref):
    qi = pl.program_id(1)
    D = HEAD_DIM
    rep = NUM_HEADS // NUM_KV_HEADS
    r = jax.lax.broadcasted_iota(jnp.int32, (BQ, BQ), 0)
    c = jax.lax.broadcasted_iota(jnp.int32, (BQ, BQ), 1)
    tri = r >= c

    for h in range(NUM_HEADS):
        g = h // rep
        qh = q_ref[0, :, h * D:(h + 1) * D] * SCALE

        def step(ki, carry, qh=qh, g=g):
            m, l, acc = carry
            kk = k_ref[0, pl.ds(ki * BQ, BQ), g * D:(g + 1) * D]
            s = jax.lax.dot_general(qh, kk, (((1,), (1,)), ((), ())),
                                    preferred_element_type=jnp.float32)
            m_new = jnp.maximum(m, s.max(axis=1, keepdims=True))
            p = jnp.exp(s - m_new)
            alpha = jnp.exp(m - m_new)
            vv = v_ref[0, pl.ds(ki * BQ, BQ), g * D:(g + 1) * D]
            acc = acc * alpha + jax.lax.dot_general(
                p, vv, (((1,), (0,)), ((), ())),
                preferred_element_type=jnp.float32)
            l = l * alpha + p.sum(axis=1, keepdims=True)
            return m_new, l, acc

        m0 = jnp.full((BQ, 1), NEG, jnp.float32)
        l0 = jnp.zeros((BQ, 1), jnp.float32)
        acc0 = jnp.zeros((BQ, D), jnp.float32)
        m, l, acc = jax.lax.fori_loop(0, qi, step, (m0, l0, acc0))

        kk = k_ref[0, pl.ds(qi * BQ, BQ), g * D:(g + 1) * D]
        s = jax.lax.dot_general(qh, kk, (((1,), (1,)), ((), ())),
                                preferred_element_type=jnp.float32)
        s = jnp.where(tri, s, NEG)
        m_new = jnp.maximum(m, s.max(axis=1, keepdims=True))
        p = jnp.exp(s - m_new)
        alpha = jnp.exp(m - m_new)
        vv = v_ref[0, pl.ds(qi * BQ, BQ), g * D:(g + 1) * D]
        acc = acc * alpha + jax.lax.dot_general(
            p, vv, (((1,), (0,)), ((), ())),
            preferred_element_type=jnp.float32)
        l = l * alpha + p.sum(axis=1, keepdims=True)

        o_ref[0, :, h * D:(h + 1) * D] = acc / l


def kernel(q, k, v, cu_seqlens_q):
    B = int(cu_seqlens_q.shape[0]) - 1
    T = q.shape[0]
    S = T // B
    nq = S // BQ
    HD = NUM_HEADS * HEAD_DIM
    GD = NUM_KV_HEADS * HEAD_DIM

    qr = q.reshape(B, S, HD)
    kr = k.reshape(B, S, GD)
    vr = v.reshape(B, S, GD)

    ob = pl.pallas_call(
        _flash_body,
        grid=(B, nq),
        in_specs=[
            pl.BlockSpec((1, BQ, HD), lambda b, i: (b, i, 0)),
            pl.BlockSpec((1, S, GD), lambda b, i: (b, 0, 0)),
            pl.BlockSpec((1, S, GD), lambda b, i: (b, 0, 0)),
        ],
        out_specs=pl.BlockSpec((1, BQ, HD), lambda b, i: (b, i, 0)),
        out_shape=jax.ShapeDtypeStruct((B, S, HD), jnp.float32),
        compiler_params=pltpu.CompilerParams(
            dimension_semantics=("parallel", "arbitrary")),
    )(qr, kr, vr)

    return ob.reshape(T, NUM_HEADS, HEAD_DIM)
